# baseline (speedup 1.0000x reference)
import functools

import jax
import jax.numpy as jnp
from jax import lax
from jax.experimental import pallas as pl
from jax.experimental.pallas import tpu as pltpu
from jax.experimental.pallas import tpu_sc as plsc

S = 16384
H = 4096
E = 64
K = 8
G = 8
GS = 8
TG = 4
SCALE = 2.5

CHUNKS = (8192, 8192)
BS = 512
NW = 32

_NEG_INF = float("-inf")


def _mm_body(h_ref, w_ref, o_ref):
    h = h_ref[...]
    w = w_ref[...]
    logits = lax.dot_general(w, h, (((1,), (1,)), ((), ())),
                             preferred_element_type=jnp.float32)
    o_ref[...] = 1.0 / (1.0 + jnp.exp(-logits))


def _make_mm_call(start, sch):
    nblk = sch // BS
    blk0 = start // BS
    return pl.pallas_call(
        _mm_body,
        grid=(nblk,),
        in_specs=[
            pl.BlockSpec((BS, H), lambda i: (blk0 + i, 0)),
            pl.BlockSpec((E, H), lambda i: (0, 0)),
        ],
        out_specs=pl.BlockSpec((E, BS), lambda i: (0, i)),
        out_shape=jax.ShapeDtypeStruct((E, sch), jnp.float32),
        compiler_params=pltpu.CompilerParams(
            dimension_semantics=("parallel",)),
    )


def _sc_route_body(scores_hbm, bias_hbm, outw_hbm, outi_hbm,
                   svt, pv, sfc, biasv, outw, outi, *, tok_per_w, nb):
    wid = lax.axis_index("s") * 2 + lax.axis_index("c")
    base = wid * tok_per_w
    pltpu.sync_copy(scores_hbm.at[:, pl.ds(base, tok_per_w)], svt)
    pltpu.sync_copy(bias_hbm, biasv)
    iota16 = lax.iota(jnp.int32, 16)

    def batch_body(b):
        tok = b * 16
        boff = b * (E * 16)

        gsum = []
        for g in range(G):
            m1 = jnp.full((16,), _NEG_INF, jnp.float32)
            m2 = jnp.full((16,), _NEG_INF, jnp.float32)
            for j in range(GS):
                e = g * GS + j
                p = svt[e, pl.ds(tok, 16)]
                pv[pl.ds(boff + e * 16, 16)] = p
                x = p + biasv[pl.ds(e * 16, 16)]
                sfc[pl.ds(boff + e * 16, 16)] = x
                m2 = jnp.maximum(m2, jnp.minimum(m1, x))
                m1 = jnp.maximum(m1, x)
            gsum.append(m1 + m2)

        sel = []
        for g in range(G):
            cnt = jnp.zeros((16,), jnp.int32)
            for h in range(G):
                if h == g:
                    continue
                if h < g:
                    beat = gsum[h] >= gsum[g]
                else:
                    beat = gsum[h] > gsum[g]
                cnt = cnt + jnp.where(beat, 1, 0)
            sel.append(cnt < TG)

        vals = [jnp.full((16,), _NEG_INF, jnp.float32) for _ in range(K)]
        idxs = [jnp.zeros((16,), jnp.int32) for _ in range(K)]
        for e in range(E):
            t = jnp.where(sel[e // GS], sfc[pl.ds(boff + e * 16, 16)], 0.0)
            ev = jnp.full((16,), e, jnp.int32)
            c = [t > vals[j] for j in range(K)]
            nv = [jnp.where(c[0], t, vals[0])]
            ni = [jnp.where(c[0], ev, idxs[0])]
            for j in range(1, K):
                shv = jnp.where(c[j - 1], vals[j - 1], t)
                shi = jnp.where(c[j - 1], idxs[j - 1], ev)
                nv.append(jnp.where(c[j], shv, vals[j]))
                ni.append(jnp.where(c[j], shi, idxs[j]))
            vals, idxs = nv, ni

        ps = [plsc.load_gather(pv, [boff + idxs[k] * 16 + iota16])
              for k in range(K)]
        denom = ps[0]
        for k in range(1, K):
            denom = denom + ps[k]
        scale = SCALE / (denom + 1e-20)
        outbase = (tok + iota16) * K
        for k in range(K):
            plsc.store_scatter(outw, [outbase + k], ps[k] * scale)
            plsc.store_scatter(outi, [outbase + k], idxs[k])

    plsc.parallel_loop(0, nb, 1, unroll=2)(batch_body)
    pltpu.sync_copy(outw, outw_hbm.at[pl.ds(base * K, tok_per_w * K)])
    pltpu.sync_copy(outi, outi_hbm.at[pl.ds(base * K, tok_per_w * K)])


def _make_sc_call(sch):
    tok_per_w = sch // NW
    nb = tok_per_w // 16
    mesh = plsc.VectorSubcoreMesh(core_axis_name="c", subcore_axis_name="s")
    body = functools.partial(_sc_route_body, tok_per_w=tok_per_w, nb=nb)
    return functools.partial(
        pl.kernel,
        mesh=mesh,
        out_type=[
            jax.ShapeDtypeStruct((sch * K,), jnp.float32),
            jax.ShapeDtypeStruct((sch * K,), jnp.int32),
        ],
        scratch_types=[
            pltpu.VMEM((E, tok_per_w), jnp.float32),
            pltpu.VMEM((nb * E * 16,), jnp.float32),
            pltpu.VMEM((nb * E * 16,), jnp.float32),
            pltpu.VMEM((E * 16,), jnp.float32),
            pltpu.VMEM((tok_per_w * K,), jnp.float32),
            pltpu.VMEM((tok_per_w * K,), jnp.int32),
        ],
        compiler_params=pltpu.CompilerParams(needs_layout_passes=False),
    )(body)


@jax.jit
def kernel(hidden_states, weight, e_score_correction_bias):
    biasb = jnp.broadcast_to(
        e_score_correction_bias[:, None], (E, 16)).reshape(E * 16)
    tws, tis = [], []
    start = 0
    for sch in CHUNKS:
        scores_t = _make_mm_call(start, sch)(hidden_states, weight)
        topw, topi = _make_sc_call(sch)(scores_t, biasb)
        tws.append(topw.reshape(sch, K))
        tis.append(topi.reshape(sch, K))
        start += sch
    return (jnp.concatenate(tws, axis=0), jnp.concatenate(tis, axis=0))


# speedup vs baseline: 1.1232x; 1.1232x over previous
import functools

import jax
import jax.numpy as jnp
from jax import lax
from jax.experimental import pallas as pl
from jax.experimental.pallas import tpu as pltpu
from jax.experimental.pallas import tpu_sc as plsc

S = 16384
H = 4096
E = 64
K = 8
G = 8
GS = 8
TG = 4
SCALE = 2.5

CHUNKS = (8192, 8192)
BS = 512
NW = 32

_NEG_INF = float("-inf")


def _mm_body(h_ref, w_ref, o_ref):
    h = h_ref[...]
    w = w_ref[...]
    logits = lax.dot_general(w, h, (((1,), (1,)), ((), ())),
                             preferred_element_type=jnp.float32)
    o_ref[...] = 1.0 / (1.0 + jnp.exp(-logits))


def _make_mm_call(start, sch):
    nblk = sch // BS
    blk0 = start // BS
    return pl.pallas_call(
        _mm_body,
        grid=(nblk,),
        in_specs=[
            pl.BlockSpec((BS, H), lambda i: (blk0 + i, 0)),
            pl.BlockSpec((E, H), lambda i: (0, 0)),
        ],
        out_specs=pl.BlockSpec((E, BS), lambda i: (0, i)),
        out_shape=jax.ShapeDtypeStruct((E, sch), jnp.float32),
        compiler_params=pltpu.CompilerParams(
            dimension_semantics=("parallel",)),
    )


def _sc_route_body(scores_hbm, bias_hbm, outw_hbm, outi_hbm,
                   svt, pv, sfc, biasv, outw, outi, *, tok_per_w, nb):
    wid = lax.axis_index("s") * 2 + lax.axis_index("c")
    base = wid * tok_per_w
    pltpu.sync_copy(scores_hbm.at[:, pl.ds(base, tok_per_w)], svt)
    pltpu.sync_copy(bias_hbm, biasv)
    iota16 = lax.iota(jnp.int32, 16)

    def batch_body(b, _):
        tok = b * 16
        boff = 0

        gsum = []
        for g in range(G):
            m1 = jnp.full((16,), _NEG_INF, jnp.float32)
            m2 = jnp.full((16,), _NEG_INF, jnp.float32)
            for j in range(GS):
                e = g * GS + j
                p = svt[e, pl.ds(tok, 16)]
                pv[pl.ds(boff + e * 16, 16)] = p
                x = p + biasv[pl.ds(e * 16, 16)]
                sfc[pl.ds(boff + e * 16, 16)] = x
                m2 = jnp.maximum(m2, jnp.minimum(m1, x))
                m1 = jnp.maximum(m1, x)
            gsum.append(m1 + m2)

        sel = []
        for g in range(G):
            cnt = jnp.zeros((16,), jnp.int32)
            for h in range(G):
                if h == g:
                    continue
                if h < g:
                    beat = gsum[h] >= gsum[g]
                else:
                    beat = gsum[h] > gsum[g]
                cnt = cnt + jnp.where(beat, 1, 0)
            sel.append(cnt < TG)

        vals = [jnp.full((16,), _NEG_INF, jnp.float32) for _ in range(K)]
        idxs = [jnp.zeros((16,), jnp.int32) for _ in range(K)]
        for e in range(E):
            t = jnp.where(sel[e // GS], sfc[pl.ds(boff + e * 16, 16)], 0.0)
            ev = jnp.full((16,), e, jnp.int32)
            c = [t > vals[j] for j in range(K)]
            nv = [jnp.where(c[0], t, vals[0])]
            ni = [jnp.where(c[0], ev, idxs[0])]
            for j in range(1, K):
                shv = jnp.where(c[j - 1], vals[j - 1], t)
                shi = jnp.where(c[j - 1], idxs[j - 1], ev)
                nv.append(jnp.where(c[j], shv, vals[j]))
                ni.append(jnp.where(c[j], shi, idxs[j]))
            vals, idxs = nv, ni

        ps = [plsc.load_gather(pv, [boff + idxs[k] * 16 + iota16])
              for k in range(K)]
        denom = ps[0]
        for k in range(1, K):
            denom = denom + ps[k]
        scale = SCALE / (denom + 1e-20)
        outbase = (tok + iota16) * K
        for k in range(K):
            plsc.store_scatter(outw, [outbase + k], ps[k] * scale)
            plsc.store_scatter(outi, [outbase + k], idxs[k])
        return 0

    lax.fori_loop(0, nb, batch_body, 0)
    pltpu.sync_copy(outw, outw_hbm.at[pl.ds(base * K, tok_per_w * K)])
    pltpu.sync_copy(outi, outi_hbm.at[pl.ds(base * K, tok_per_w * K)])


def _make_sc_call(sch):
    tok_per_w = sch // NW
    nb = tok_per_w // 16
    mesh = plsc.VectorSubcoreMesh(core_axis_name="c", subcore_axis_name="s")
    body = functools.partial(_sc_route_body, tok_per_w=tok_per_w, nb=nb)
    return functools.partial(
        pl.kernel,
        mesh=mesh,
        out_type=[
            jax.ShapeDtypeStruct((sch * K,), jnp.float32),
            jax.ShapeDtypeStruct((sch * K,), jnp.int32),
        ],
        scratch_types=[
            pltpu.VMEM((E, tok_per_w), jnp.float32),
            pltpu.VMEM((E * 16,), jnp.float32),
            pltpu.VMEM((E * 16,), jnp.float32),
            pltpu.VMEM((E * 16,), jnp.float32),
            pltpu.VMEM((tok_per_w * K,), jnp.float32),
            pltpu.VMEM((tok_per_w * K,), jnp.int32),
        ],
        compiler_params=pltpu.CompilerParams(needs_layout_passes=False),
    )(body)


@jax.jit
def kernel(hidden_states, weight, e_score_correction_bias):
    biasb = jnp.broadcast_to(
        e_score_correction_bias[:, None], (E, 16)).reshape(E * 16)
    tws, tis = [], []
    start = 0
    for sch in CHUNKS:
        scores_t = _make_mm_call(start, sch)(hidden_states, weight)
        topw, topi = _make_sc_call(sch)(scores_t, biasb)
        tws.append(topw.reshape(sch, K))
        tis.append(topi.reshape(sch, K))
        start += sch
    return (jnp.concatenate(tws, axis=0), jnp.concatenate(tis, axis=0))
